# baseline (speedup 1.0000x reference)
import jax
import jax.numpy as jnp
from jax.experimental import pallas as pl
from jax.experimental.pallas import tpu as pltpu

N = 4096
D = 256
SB = 512
NSB = N // SB
HW = N // 2

_FIRST_HALF = (
    (9, 0, 1024),
    (11, 1024, 2048),
    (13, 2048, 3072),
    (15, 3072, 3584),
    (16, 3584, 4096),
)
_SECOND_HALF = (
    (17, 0, 2048),
    (18, 2048, 4096),
)


def _fused_gcn_kernel(adj_ref, x_ref, w1_ref, b1_ref, w2_ref, b2_ref,
                      o_ref, adjbf_ref, s2_ref):
    m = pl.program_id(0)

    for c, r0, r1 in _FIRST_HALF:
        @pl.when(m == c)
        def _(r0=r0, r1=r1):
            o_ref[r0:r1, :] = jnp.broadcast_to(
                b2_ref[...], (r1 - r0, D)
            ) + jnp.dot(
                adjbf_ref[r0:r1, :HW], s2_ref[:HW, :],
                preferred_element_type=jnp.float32,
            )

    for c, r0, r1 in _SECOND_HALF:
        @pl.when(m == c)
        def _(r0=r0, r1=r1):
            o_ref[r0:r1, :] += jnp.dot(
                adjbf_ref[r0:r1, HW:], s2_ref[HW:, :],
                preferred_element_type=jnp.float32,
            )

    @pl.when(jnp.logical_and(m >= 2, jnp.logical_and(m <= 2 * NSB,
                                                     m % 2 == 0)))
    def _():
        b0 = (m - 2) // 2 * SB
        arow = adjbf_ref[pl.ds(b0, SB), :]
        u = jnp.dot(arow, x_ref[...], preferred_element_type=jnp.float32)
        t = jnp.dot(
            u.astype(jnp.bfloat16), w1_ref[...],
            preferred_element_type=jnp.float32,
        )
        h = jnp.maximum(t + b1_ref[...], 0.0).astype(jnp.bfloat16)
        s2_ref[pl.ds(b0, SB), :] = jnp.dot(
            h, w2_ref[...], preferred_element_type=jnp.float32
        ).astype(jnp.bfloat16)

    @pl.when(m < 2 * NSB)
    def _():
        adjbf_ref[pl.ds(m // 2 * SB, SB),
                  pl.ds(m % 2 * HW, HW)] = adj_ref[...].astype(jnp.bfloat16)


def kernel(x, adj, W1, b1, W2, b2):
    xb = x.astype(jnp.bfloat16)
    w1b = W1.astype(jnp.bfloat16)
    w2b = W2.astype(jnp.bfloat16)
    b1r = b1.reshape(1, D)
    b2r = b2.reshape(1, D)
    return pl.pallas_call(
        _fused_gcn_kernel,
        grid=(2 * NSB + 3,),
        in_specs=[
            pl.BlockSpec(
                (SB, HW),
                lambda i: (jnp.minimum(i, 2 * NSB - 1) // 2,
                           jnp.minimum(i, 2 * NSB - 1) % 2),
            ),
            pl.BlockSpec((N, D), lambda i: (0, 0)),
            pl.BlockSpec((D, D), lambda i: (0, 0)),
            pl.BlockSpec((1, D), lambda i: (0, 0)),
            pl.BlockSpec((D, D), lambda i: (0, 0)),
            pl.BlockSpec((1, D), lambda i: (0, 0)),
        ],
        out_specs=pl.BlockSpec((N, D), lambda i: (0, 0)),
        out_shape=jax.ShapeDtypeStruct((N, D), jnp.float32),
        scratch_shapes=[
            pltpu.VMEM((N, N), jnp.bfloat16),
            pltpu.VMEM((N, D), jnp.bfloat16),
        ],
    )(adj, xb, w1b, b1r, w2b, b2r)


# speedup vs baseline: 1.1459x; 1.1459x over previous
import jax
import jax.numpy as jnp
from jax.experimental import pallas as pl
from jax.experimental.pallas import tpu as pltpu

N = 4096
D = 256
BM = 512
NB = N // BM


def _fused_gcn_kernel(adj_ref, x_ref, w1_ref, b1_ref, w2_ref, b2_ref,
                      o_ref, adjbf_ref, s_ref, h_ref):
    i = pl.program_id(0)

    @pl.when(i == 0)
    def _():
        s_ref[...] = jnp.dot(
            x_ref[...], w1_ref[...], preferred_element_type=jnp.float32
        ).astype(jnp.bfloat16)

    @pl.when(i < NB)
    def _():
        ab = adj_ref[...].astype(jnp.bfloat16)
        adjbf_ref[pl.ds(i * BM, BM), :] = ab
        t = jnp.dot(ab, s_ref[...], preferred_element_type=jnp.float32)
        h_ref[pl.ds(i * BM, BM), :] = jnp.maximum(
            t + b1_ref[...], 0.0
        ).astype(jnp.bfloat16)

    @pl.when(i == NB)
    def _():
        s_ref[...] = jnp.dot(
            h_ref[...], w2_ref[...], preferred_element_type=jnp.float32
        ).astype(jnp.bfloat16)

    @pl.when(i >= NB)
    def _():
        ab = adjbf_ref[pl.ds((i - NB) * BM, BM), :]
        o_ref[...] = (
            jnp.dot(ab, s_ref[...], preferred_element_type=jnp.float32)
            + b2_ref[...]
        )


def kernel(x, adj, W1, b1, W2, b2):
    xb = x.astype(jnp.bfloat16)
    w1b = W1.astype(jnp.bfloat16)
    w2b = W2.astype(jnp.bfloat16)
    b1r = b1.reshape(1, D)
    b2r = b2.reshape(1, D)
    return pl.pallas_call(
        _fused_gcn_kernel,
        grid=(2 * NB,),
        in_specs=[
            pl.BlockSpec((BM, N), lambda i: (jnp.minimum(i, NB - 1), 0)),
            pl.BlockSpec((N, D), lambda i: (0, 0)),
            pl.BlockSpec((D, D), lambda i: (0, 0)),
            pl.BlockSpec((1, D), lambda i: (0, 0)),
            pl.BlockSpec((D, D), lambda i: (0, 0)),
            pl.BlockSpec((1, D), lambda i: (0, 0)),
        ],
        out_specs=pl.BlockSpec((BM, D), lambda i: (jnp.maximum(i - NB, 0), 0)),
        out_shape=jax.ShapeDtypeStruct((N, D), jnp.float32),
        scratch_shapes=[
            pltpu.VMEM((N, N), jnp.bfloat16),
            pltpu.VMEM((N, D), jnp.bfloat16),
            pltpu.VMEM((N, D), jnp.bfloat16),
        ],
    )(adj, xb, w1b, b1r, w2b, b2r)
